# Initial kernel scaffold; baseline (speedup 1.0000x reference)
import jax
import jax.numpy as jnp
from jax.experimental import pallas as pl


def kernel(inp, Wg, bg, W1, b1, W2, b2):
    raise NotImplementedError("write your pallas kernel here")


# speedup vs baseline: 1.6957x; 1.6957x over previous
import functools

import jax
import jax.numpy as jnp
from jax.experimental import pallas as pl
from jax.experimental.pallas import tpu as pltpu


def _moe_dense_body(x_ref, wg_ref, bg_ref, w1_ref, b1_ref, w2_ref, b2_ref,
                    out_ref, w_s, y_s, *, nh, topk):
    e = pl.program_id(1)
    h = pl.program_id(2)

    @pl.when(jnp.logical_and(e == 0, h == 0))
    def _gate():
        x = x_ref[...]
        logits = jnp.dot(x, wg_ref[...], preferred_element_type=jnp.float32)
        logits = logits + bg_ref[...]
        tb, ne = logits.shape
        iota_e = jax.lax.broadcasted_iota(jnp.int32, (tb, ne), 1)
        m1 = jnp.max(logits, axis=1, keepdims=True)
        a1 = jnp.min(jnp.where(logits == m1, iota_e, ne), axis=1, keepdims=True)
        l2 = jnp.where(iota_e == a1, -jnp.inf, logits)
        m2 = jnp.max(l2, axis=1, keepdims=True)
        a2 = jnp.min(jnp.where(l2 == m2, iota_e, ne), axis=1, keepdims=True)
        g1 = 1.0 / (1.0 + jnp.exp(m2 - m1))
        g2 = 1.0 - g1
        w_s[...] = (jnp.where(iota_e == a1, g1, 0.0)
                    + jnp.where(iota_e == a2, g2, 0.0))

    @pl.when(h == 0)
    def _init_acc():
        y_s[...] = jnp.zeros_like(y_s)

    x = x_ref[...]
    hid = jnp.dot(x, w1_ref[0], preferred_element_type=jnp.float32)
    hid = jax.nn.gelu(hid + b1_ref[0])
    y_s[...] += jnp.dot(hid, w2_ref[0], preferred_element_type=jnp.float32)

    @pl.when(h == nh - 1)
    def _combine():
        w_all = w_s[...]
        iota_e = jax.lax.broadcasted_iota(jnp.int32, w_all.shape, 1)
        we = jnp.sum(jnp.where(iota_e == e, w_all, 0.0), axis=1, keepdims=True)
        contrib = (y_s[...] + b2_ref[0]) * we

        @pl.when(e == 0)
        def _():
            out_ref[...] = contrib

        @pl.when(e > 0)
        def _():
            out_ref[...] += contrib


def kernel(inp, Wg, bg, W1, b1, W2, b2):
    T, D = inp.shape
    E = Wg.shape[1]
    H = W1.shape[2]
    K = 2

    TB = min(512, T)
    HT = min(512, H)
    nt, nh = T // TB, H // HT

    grid = (nt, E, nh)
    out = pl.pallas_call(
        functools.partial(_moe_dense_body, nh=nh, topk=K),
        grid=grid,
        in_specs=[
            pl.BlockSpec((TB, D), lambda t, e, h: (t, 0)),
            pl.BlockSpec((D, E), lambda t, e, h: (0, 0)),
            pl.BlockSpec((1, E), lambda t, e, h: (0, 0)),
            pl.BlockSpec((1, D, HT), lambda t, e, h: (e, 0, h)),
            pl.BlockSpec((1, 1, HT), lambda t, e, h: (e, 0, h)),
            pl.BlockSpec((1, HT, D), lambda t, e, h: (e, h, 0)),
            pl.BlockSpec((1, 1, D), lambda t, e, h: (e, 0, 0)),
        ],
        out_specs=pl.BlockSpec((TB, D), lambda t, e, h: (t, 0)),
        out_shape=jax.ShapeDtypeStruct((T, D), jnp.float32),
        scratch_shapes=[
            pltpu.VMEM((TB, E), jnp.float32),
            pltpu.VMEM((TB, D), jnp.float32),
        ],
        compiler_params=pltpu.CompilerParams(
            dimension_semantics=("parallel", "arbitrary", "arbitrary"),
        ),
    )(inp, Wg, bg.reshape(1, E), W1, b1.reshape(E, 1, H), W2,
      b2.reshape(E, 1, D))
    return out


# speedup vs baseline: 3.3416x; 1.9706x over previous
import functools

import jax
import jax.numpy as jnp
from jax import lax
from jax.experimental import pallas as pl
from jax.experimental.pallas import tpu as pltpu
from jax.experimental.pallas import tpu_sc as plsc

BLK = 512
CH = 512
HT = 512
CS = 64


def _gate_route_body(x_ref, wg_ref, bg_ref, dest_ref, gw_ref, be_ref, *,
                     blk, nb):
    T, _ = x_ref.shape
    E = wg_ref.shape[1]
    x = x_ref[...]
    logits = jnp.dot(x, wg_ref[...], preferred_element_type=jnp.float32)
    logits = logits + bg_ref[...]
    iota_e = lax.broadcasted_iota(jnp.int32, (T, E), 1)
    m1 = jnp.max(logits, axis=1, keepdims=True)
    a1 = jnp.min(jnp.where(logits == m1, iota_e, E), axis=1, keepdims=True)
    l2 = jnp.where(iota_e == a1, -jnp.inf, logits)
    m2 = jnp.max(l2, axis=1, keepdims=True)
    a2 = jnp.min(jnp.where(l2 == m2, iota_e, E), axis=1, keepdims=True)
    g1 = 1.0 / (1.0 + jnp.exp(m2 - m1))
    gw_ref[...] = jnp.concatenate([g1, 1.0 - g1], axis=0)

    e_pair = jnp.concatenate([a1, a2], axis=0)
    P = 2 * T
    iota_pe = lax.broadcasted_iota(jnp.int32, (P, E), 1)
    onehot = (iota_pe == e_pair).astype(jnp.float32)

    counts = jnp.sum(onehot, axis=0, keepdims=True)
    pc = jnp.ceil(counts / blk) * blk
    off_cols = []
    run = jnp.zeros((1, 1), jnp.float32)
    for e in range(E):
        off_cols.append(run)
        run = run + lax.slice(pc, (0, e), (1, e + 1))
    off = jnp.concatenate(off_cols, axis=1)

    bidx = lax.broadcasted_iota(jnp.int32, (nb, E), 0).astype(jnp.float32) * blk
    be = jnp.sum((bidx >= off).astype(jnp.int32), axis=1, keepdims=True) - 1
    be_ref[...] = be

    ic = lax.broadcasted_iota(jnp.int32, (CH, CH), 0)
    jc = lax.broadcasted_iota(jnp.int32, (CH, CH), 1)
    tril = (jc < ic).astype(jnp.float32)
    carry = jnp.zeros((1, E), jnp.float32)
    for c in range(P // CH):
        oh = onehot[c * CH:(c + 1) * CH, :]
        within = jnp.dot(tril, oh, preferred_element_type=jnp.float32)
        slot = jnp.sum(oh * (within + carry + off), axis=1, keepdims=True)
        dest_ref[c * CH:(c + 1) * CH, :] = slot.astype(jnp.int32)
        carry = carry + jnp.sum(oh, axis=0, keepdims=True)


def _ffn_body(be_s, x_ref, w1_ref, b1_ref, w2_ref, b2_ref, ys_ref, acc, *,
              nh):
    h = pl.program_id(1)

    @pl.when(h == 0)
    def _():
        acc[...] = jnp.zeros_like(acc)

    hid = jnp.dot(x_ref[...], w1_ref[0], preferred_element_type=jnp.float32)
    hid = jax.nn.gelu(hid + b1_ref[0])
    acc[...] += jnp.dot(hid, w2_ref[0], preferred_element_type=jnp.float32)

    @pl.when(h == nh - 1)
    def _():
        ys_ref[...] = acc[...] + b2_ref[0]


def _combine_body(y0_ref, y1_ref, g0_ref, g1_ref, out_ref):
    out_ref[...] = y0_ref[...] * g0_ref[...] + y1_ref[...] * g1_ref[...]


def kernel(inp, Wg, bg, W1, b1, W2, b2):
    T, D = inp.shape
    E = Wg.shape[1]
    H = W1.shape[2]
    P = 2 * T
    S = P + E * BLK
    NB = S // BLK
    NH = H // HT

    dest2, gw, be = pl.pallas_call(
        functools.partial(_gate_route_body, blk=BLK, nb=NB),
        out_shape=(
            jax.ShapeDtypeStruct((P, 1), jnp.int32),
            jax.ShapeDtypeStruct((P, 1), jnp.float32),
            jax.ShapeDtypeStruct((NB, 1), jnp.int32),
        ),
    )(inp, Wg, bg.reshape(1, E))
    dest_flat = dest2.reshape(P)

    info = plsc.get_sparse_core_info()
    NC, NS = info.num_cores, info.num_subcores
    NW = NC * NS
    pairs_w = P // NW
    nch = pairs_w // CS
    mesh = plsc.VectorSubcoreMesh(core_axis_name="c", subcore_axis_name="s")

    @functools.partial(
        pl.kernel,
        out_type=jax.ShapeDtypeStruct((S, D), jnp.float32),
        mesh=mesh,
        scratch_types=[
            pltpu.VMEM((CS,), jnp.int32),
            pltpu.VMEM((CS, D), jnp.float32),
            pltpu.SemaphoreType.DMA,
        ],
    )
    def _scatter_k(x_hbm, dest_hbm, xs_hbm, idx_v, rows_v, sem):
        wid = lax.axis_index("s") * NC + lax.axis_index("c")
        base = wid * pairs_w

        def chunk(c, carry):
            p0 = base + c * CS
            k = p0 // T
            t0 = p0 - k * T
            pltpu.sync_copy(dest_hbm.at[pl.ds(p0, CS)], idx_v)
            pltpu.sync_copy(x_hbm.at[pl.ds(t0, CS)], rows_v)
            pltpu.async_copy(rows_v, xs_hbm.at[idx_v], sem).wait()
            return carry

        lax.fori_loop(0, nch, chunk, 0)

    xs = _scatter_k(inp, dest_flat)

    grid_spec = pltpu.PrefetchScalarGridSpec(
        num_scalar_prefetch=1,
        grid=(NB, NH),
        in_specs=[
            pl.BlockSpec((BLK, D), lambda b, h, be_s: (b, 0)),
            pl.BlockSpec((1, D, HT), lambda b, h, be_s: (be_s[b], 0, h)),
            pl.BlockSpec((1, 1, HT), lambda b, h, be_s: (be_s[b], 0, h)),
            pl.BlockSpec((1, HT, D), lambda b, h, be_s: (be_s[b], h, 0)),
            pl.BlockSpec((1, 1, D), lambda b, h, be_s: (be_s[b], 0, 0)),
        ],
        out_specs=pl.BlockSpec((BLK, D), lambda b, h, be_s: (b, 0)),
        scratch_shapes=[pltpu.VMEM((BLK, D), jnp.float32)],
    )
    ys = pl.pallas_call(
        functools.partial(_ffn_body, nh=NH),
        grid_spec=grid_spec,
        out_shape=jax.ShapeDtypeStruct((S, D), jnp.float32),
        compiler_params=pltpu.CompilerParams(
            dimension_semantics=("parallel", "arbitrary"),
        ),
    )(be.reshape(NB), xs, W1, b1.reshape(E, 1, H), W2, b2.reshape(E, 1, D))

    @functools.partial(
        pl.kernel,
        out_type=jax.ShapeDtypeStruct((P, D), jnp.float32),
        mesh=mesh,
        scratch_types=[
            pltpu.VMEM((CS,), jnp.int32),
            pltpu.VMEM((CS, D), jnp.float32),
            pltpu.SemaphoreType.DMA,
        ],
    )
    def _gather_k(ys_hbm, dest_hbm, yp_hbm, idx_v, rows_v, sem):
        wid = lax.axis_index("s") * NC + lax.axis_index("c")
        base = wid * pairs_w

        def chunk(c, carry):
            p0 = base + c * CS
            pltpu.sync_copy(dest_hbm.at[pl.ds(p0, CS)], idx_v)
            pltpu.async_copy(ys_hbm.at[idx_v], rows_v, sem).wait()
            pltpu.sync_copy(rows_v, yp_hbm.at[pl.ds(p0, CS)])
            return carry

        lax.fori_loop(0, nch, chunk, 0)

    yp = _gather_k(ys, dest_flat)

    TB = 512
    nt = T // TB
    out = pl.pallas_call(
        _combine_body,
        grid=(nt,),
        in_specs=[
            pl.BlockSpec((TB, D), lambda t: (t, 0)),
            pl.BlockSpec((TB, D), lambda t: (t + nt, 0)),
            pl.BlockSpec((TB, 1), lambda t: (t, 0)),
            pl.BlockSpec((TB, 1), lambda t: (t + nt, 0)),
        ],
        out_specs=pl.BlockSpec((TB, D), lambda t: (t, 0)),
        out_shape=jax.ShapeDtypeStruct((T, D), jnp.float32),
        compiler_params=pltpu.CompilerParams(
            dimension_semantics=("parallel",),
        ),
    )(yp, yp, gw, gw)
    return out
